# Initial kernel scaffold; baseline (speedup 1.0000x reference)
import jax
import jax.numpy as jnp
from jax.experimental import pallas as pl


def kernel(x, edge_index, edge_weight, W1, b1, W2, b2):
    raise NotImplementedError("write your pallas kernel here")


# speedup vs baseline: 3.8996x; 3.8996x over previous
import functools

import jax
import jax.numpy as jnp
from jax import lax
from jax.experimental import pallas as pl
from jax.experimental.pallas import tpu as pltpu
from jax.experimental.pallas import tpu_sc as plsc

NC = 2
NS = 16
NW = NC * NS
LANES = 16
C = 96


def _mm_tc(x, w, block_rows=1000):
    n, kdim = x.shape
    m = w.shape[1]

    def body(x_ref, w_ref, o_ref):
        o_ref[...] = jnp.dot(x_ref[...], w_ref[...],
                             preferred_element_type=jnp.float32)

    return pl.pallas_call(
        body,
        grid=(n // block_rows,),
        in_specs=[pl.BlockSpec((block_rows, kdim), lambda i: (i, 0)),
                  pl.BlockSpec((kdim, m), lambda i: (0, 0))],
        out_specs=pl.BlockSpec((block_rows, m), lambda i: (i, 0)),
        out_shape=jax.ShapeDtypeStruct((n, m), jnp.float32),
    )(x, w)


def _bias_relu_mm_tc(parts, b, w, block_rows=1000):
    _, n, kdim = parts.shape
    m = w.shape[1]

    def body(p_ref, b_ref, w_ref, o_ref):
        h = jnp.maximum(p_ref[0] + p_ref[1] + b_ref[...], 0.0)
        o_ref[...] = jnp.dot(h, w_ref[...], preferred_element_type=jnp.float32)

    return pl.pallas_call(
        body,
        grid=(n // block_rows,),
        in_specs=[pl.BlockSpec((2, block_rows, kdim), lambda i: (0, i, 0)),
                  pl.BlockSpec((1, kdim), lambda i: (0, 0)),
                  pl.BlockSpec((kdim, m), lambda i: (0, 0))],
        out_specs=pl.BlockSpec((block_rows, m), lambda i: (i, 0)),
        out_shape=jax.ShapeDtypeStruct((n, m), jnp.float32),
    )(parts, b.reshape(1, kdim), w)


def _bias_relu_tc(parts, b, block_rows=1000):
    _, n, mp = parts.shape
    m = b.shape[0]

    def body(p_ref, b_ref, o_ref):
        o_ref[...] = jnp.maximum(
            p_ref[0, :, :m] + p_ref[1, :, :m] + b_ref[...], 0.0)

    return pl.pallas_call(
        body,
        grid=(n // block_rows,),
        in_specs=[pl.BlockSpec((2, block_rows, mp), lambda i: (0, i, 0)),
                  pl.BlockSpec((1, m), lambda i: (0, 0))],
        out_specs=pl.BlockSpec((block_rows, m), lambda i: (i, 0)),
        out_shape=jax.ShapeDtypeStruct((n, m), jnp.float32),
    )(parts, b.reshape(1, m))


def _sc_spmm(src, dst, ew, support, n_out):
    e_pad = src.shape[0]
    f = support.shape[1]
    epw = e_pad // NW
    nchunk = epw // C
    fb_n = f // LANES
    zrows = 80
    full = ((n_out + NS - 1) // NS + zrows - 1) // zrows * zrows
    last = n_out - (NS - 1) * full
    assert 0 < last <= full and last % zrows == 0

    mesh = plsc.VectorSubcoreMesh(core_axis_name="c", subcore_axis_name="s",
                                  num_cores=NC, num_subcores=NS)

    @functools.partial(
        pl.kernel,
        out_type=jax.ShapeDtypeStruct((NC, n_out, f), jnp.float32),
        mesh=mesh,
        scratch_types=[
            pltpu.VMEM((C, f), jnp.float32),
            pltpu.VMEM((C, f), jnp.float32),
            pltpu.VMEM((C, f), jnp.float32),
            pltpu.VMEM((C, f), jnp.float32),
            pltpu.VMEM((C,), jnp.int32),
            pltpu.VMEM((C,), jnp.int32),
            pltpu.VMEM((C,), jnp.int32),
            pltpu.VMEM((C,), jnp.int32),
            pltpu.VMEM((C + LANES,), jnp.float32),
            pltpu.VMEM((C + LANES,), jnp.float32),
            pltpu.VMEM_SHARED((n_out, f), jnp.float32),
            pltpu.SemaphoreType.DMA,
            pltpu.SemaphoreType.DMA,
            pltpu.SemaphoreType.DMA,
            pltpu.SemaphoreType.DMA,
            pltpu.SemaphoreType.DMA,
            pltpu.SemaphoreType.DMA,
            pltpu.SemaphoreType.DMA,
            pltpu.SemaphoreType.DMA,
            pltpu.SemaphoreType.DMA,
            pltpu.SemaphoreType.DMA,
        ],
    )
    def spmm(src_hbm, dst_hbm, w_hbm, sup_hbm, out_hbm,
             rows0, rows1, sc0, sc1, si0, si1, di0, di1, wv0, wv1,
             acc, g0, g1, s0, s1, d0, d1, r0sem, r1sem, w0, w1):
        cid = lax.axis_index("c")
        sid = lax.axis_index("s")
        wid = cid * NS + sid
        eb = wid * epw
        rows = (rows0, rows1)
        scaled = (sc0, sc1)
        srcs = (si0, si1)
        dsts = (di0, di1)
        wvs = (wv0, wv1)
        gsem = (g0, g1)
        ssem = (s0, s1)
        dsem = (d0, d1)
        rsem = (r0sem, r1sem)
        wsem = (w0, w1)
        nkk = nchunk // 2

        def zrow(i, carry):
            for fb in range(fb_n):
                rows0[i, pl.ds(fb * LANES, LANES)] = jnp.zeros((LANES,),
                                                               jnp.float32)
            return carry
        lax.fori_loop(0, zrows, zrow, 0)
        for j in range(full // zrows):
            @pl.when(jnp.logical_or(sid < NS - 1, j < last // zrows))
            def _():
                pltpu.sync_copy(
                    rows0.at[pl.ds(0, zrows)],
                    acc.at[pl.ds(sid * full + j * zrows, zrows)])
        plsc.subcore_barrier()

        for b in range(2):
            pltpu.sync_copy(src_hbm.at[pl.ds(eb + b * C, C)], srcs[b])
            pltpu.async_copy(sup_hbm.at[srcs[b]], rows[b], gsem[b])
            pltpu.async_copy(w_hbm.at[pl.ds(eb + b * C, C)],
                             wvs[b].at[pl.ds(0, C)], wsem[b])

        def chunk_step(kk, carry):
            for b in range(2):
                k = kk * 2 + b
                pltpu.make_async_copy(sup_hbm.at[srcs[b]], rows[b],
                                      gsem[b]).wait()

                @pl.when(kk < nkk - 1)
                def _():
                    pltpu.async_copy(src_hbm.at[pl.ds(eb + (k + 2) * C, C)],
                                     srcs[b], rsem[b])

                @pl.when(kk >= 1)
                def _():
                    pltpu.make_async_copy(
                        scaled[b], acc.at[dsts[b]], ssem[b]).wait()
                pltpu.async_copy(dst_hbm.at[pl.ds(eb + k * C, C)],
                                 dsts[b], dsem[b])

                pltpu.make_async_copy(w_hbm.at[pl.ds(eb + k * C, C)],
                                      wvs[b].at[pl.ds(0, C)], wsem[b]).wait()

                def edge(e, ecarry):
                    we = wvs[b][pl.ds(e, LANES)][0]
                    for fb in range(fb_n):
                        sl = pl.ds(fb * LANES, LANES)
                        scaled[b][e, sl] = rows[b][e, sl] * we
                    return ecarry
                lax.fori_loop(0, C, edge, 0)

                @pl.when(kk < nkk - 1)
                def _():
                    pltpu.make_async_copy(
                        src_hbm.at[pl.ds(eb + (k + 2) * C, C)], srcs[b],
                        rsem[b]).wait()
                    pltpu.async_copy(sup_hbm.at[srcs[b]], rows[b], gsem[b])
                    pltpu.async_copy(w_hbm.at[pl.ds(eb + (k + 2) * C, C)],
                                     wvs[b].at[pl.ds(0, C)], wsem[b])
                pltpu.make_async_copy(
                    dst_hbm.at[pl.ds(eb + k * C, C)], dsts[b],
                    dsem[b]).wait()
                pltpu.async_copy(scaled[b], acc.at[dsts[b]], ssem[b],
                                 add=True)
            return carry
        lax.fori_loop(0, nkk, chunk_step, 0)

        for b in range(2):
            pltpu.make_async_copy(scaled[b], acc.at[dsts[b]], ssem[b]).wait()
        plsc.subcore_barrier()
        r0 = sid * full

        @pl.when(sid < NS - 1)
        def _():
            pltpu.sync_copy(acc.at[pl.ds(r0, full)],
                            out_hbm.at[cid, pl.ds(r0, full)])

        @pl.when(sid == NS - 1)
        def _():
            pltpu.sync_copy(acc.at[pl.ds(r0, last)],
                            out_hbm.at[cid, pl.ds(r0, last)])

    return spmm(src, dst, ew, support)


def kernel(x, edge_index, edge_weight, W1, b1, W2, b2):
    n, _ = x.shape
    e = edge_weight.shape[0]

    quantum = NW * C * 2
    e_pad = ((e + quantum - 1) // quantum) * quantum
    pad = e_pad - e
    src = jnp.concatenate([edge_index[0], jnp.zeros((pad,), jnp.int32)])
    dst = jnp.concatenate([edge_index[1], jnp.zeros((pad,), jnp.int32)])
    ew = jnp.concatenate([edge_weight, jnp.zeros((pad,), jnp.float32)])

    h1, h2 = W2.shape
    W2p = jnp.concatenate([W2, jnp.zeros((h1, h1 - h2), jnp.float32)], axis=1)

    support = _mm_tc(x, W1)
    p1 = _sc_spmm(src, dst, ew, support, n)
    s2 = _bias_relu_mm_tc(p1, b1, W2p)
    p2 = _sc_spmm(src, dst, ew, s2, n)
    return _bias_relu_tc(p2, b2)


# speedup vs baseline: 5.0530x; 1.2958x over previous
import functools

import jax
import jax.numpy as jnp
from jax import lax
from jax.experimental import pallas as pl
from jax.experimental.pallas import tpu as pltpu
from jax.experimental.pallas import tpu_sc as plsc

NC = 2
NS = 16
NW = NC * NS
LANES = 16
C = 96


def _mm_tc(x, w, block_rows=1000):
    n, kdim = x.shape
    m = w.shape[1]

    def body(x_ref, w_ref, o_ref):
        o_ref[...] = jnp.dot(x_ref[...], w_ref[...],
                             preferred_element_type=jnp.float32)

    return pl.pallas_call(
        body,
        grid=(n // block_rows,),
        in_specs=[pl.BlockSpec((block_rows, kdim), lambda i: (i, 0)),
                  pl.BlockSpec((kdim, m), lambda i: (0, 0))],
        out_specs=pl.BlockSpec((block_rows, m), lambda i: (i, 0)),
        out_shape=jax.ShapeDtypeStruct((n, m), jnp.float32),
    )(x, w)


def _bias_relu_mm_tc(parts, b, w, block_rows=1000):
    _, n, kdim = parts.shape
    m = w.shape[1]

    def body(p_ref, b_ref, w_ref, o_ref):
        h = jnp.maximum(p_ref[0] + p_ref[1] + b_ref[...], 0.0)
        o_ref[...] = jnp.dot(h, w_ref[...], preferred_element_type=jnp.float32)

    return pl.pallas_call(
        body,
        grid=(n // block_rows,),
        in_specs=[pl.BlockSpec((2, block_rows, kdim), lambda i: (0, i, 0)),
                  pl.BlockSpec((1, kdim), lambda i: (0, 0)),
                  pl.BlockSpec((kdim, m), lambda i: (0, 0))],
        out_specs=pl.BlockSpec((block_rows, m), lambda i: (i, 0)),
        out_shape=jax.ShapeDtypeStruct((n, m), jnp.float32),
    )(parts, b.reshape(1, kdim), w)


def _bias_relu_tc(parts, b, block_rows=1000):
    _, n, mp = parts.shape
    m = b.shape[0]

    def body(p_ref, b_ref, o_ref):
        o_ref[...] = jnp.maximum(
            p_ref[0, :, :m] + p_ref[1, :, :m] + b_ref[...], 0.0)

    return pl.pallas_call(
        body,
        grid=(n // block_rows,),
        in_specs=[pl.BlockSpec((2, block_rows, mp), lambda i: (0, i, 0)),
                  pl.BlockSpec((1, m), lambda i: (0, 0))],
        out_specs=pl.BlockSpec((block_rows, m), lambda i: (i, 0)),
        out_shape=jax.ShapeDtypeStruct((n, m), jnp.float32),
    )(parts, b.reshape(1, m))


def _sc_spmm(src, dst, ew, support, n_out):
    e_pad = src.shape[0]
    f = support.shape[1]
    epw = e_pad // NW
    nchunk = epw // C
    fb_n = f // LANES
    zrows = 80
    full = ((n_out + NS - 1) // NS + zrows - 1) // zrows * zrows
    last = n_out - (NS - 1) * full
    assert 0 < last <= full and last % zrows == 0

    mesh = plsc.VectorSubcoreMesh(core_axis_name="c", subcore_axis_name="s",
                                  num_cores=NC, num_subcores=NS)
    params = (None if f % 128 == 0
              else pltpu.CompilerParams(use_tc_tiling_on_sc=False))

    @functools.partial(
        pl.kernel,
        out_type=jax.ShapeDtypeStruct((NC, n_out, f), jnp.float32),
        mesh=mesh,
        compiler_params=params,
        scratch_types=[
            pltpu.VMEM((C, f), jnp.float32),
            pltpu.VMEM((C, f), jnp.float32),
            pltpu.VMEM((C, f), jnp.float32),
            pltpu.VMEM((C, f), jnp.float32),
            pltpu.VMEM((C,), jnp.int32),
            pltpu.VMEM((C,), jnp.int32),
            pltpu.VMEM((C,), jnp.int32),
            pltpu.VMEM((C,), jnp.int32),
            pltpu.VMEM((C + LANES,), jnp.float32),
            pltpu.VMEM((C + LANES,), jnp.float32),
            pltpu.VMEM_SHARED((n_out, f), jnp.float32),
            pltpu.SemaphoreType.DMA,
            pltpu.SemaphoreType.DMA,
            pltpu.SemaphoreType.DMA,
            pltpu.SemaphoreType.DMA,
            pltpu.SemaphoreType.DMA,
            pltpu.SemaphoreType.DMA,
            pltpu.SemaphoreType.DMA,
            pltpu.SemaphoreType.DMA,
            pltpu.SemaphoreType.DMA,
            pltpu.SemaphoreType.DMA,
        ],
    )
    def spmm(src_hbm, dst_hbm, w_hbm, sup_hbm, out_hbm,
             rows0, rows1, sc0, sc1, si0, si1, di0, di1, wv0, wv1,
             acc, g0, g1, s0, s1, d0, d1, r0sem, r1sem, w0, w1):
        cid = lax.axis_index("c")
        sid = lax.axis_index("s")
        wid = cid * NS + sid
        eb = wid * epw
        rows = (rows0, rows1)
        scaled = (sc0, sc1)
        srcs = (si0, si1)
        dsts = (di0, di1)
        wvs = (wv0, wv1)
        gsem = (g0, g1)
        ssem = (s0, s1)
        dsem = (d0, d1)
        rsem = (r0sem, r1sem)
        wsem = (w0, w1)
        nkk = nchunk // 2

        def zrow(i, carry):
            for fb in range(fb_n):
                rows0[i, pl.ds(fb * LANES, LANES)] = jnp.zeros((LANES,),
                                                               jnp.float32)
            return carry
        lax.fori_loop(0, zrows, zrow, 0)
        for j in range(full // zrows):
            @pl.when(jnp.logical_or(sid < NS - 1, j < last // zrows))
            def _():
                pltpu.sync_copy(
                    rows0.at[pl.ds(0, zrows)],
                    acc.at[pl.ds(sid * full + j * zrows, zrows)])
        plsc.subcore_barrier()

        for b in range(2):
            pltpu.sync_copy(src_hbm.at[pl.ds(eb + b * C, C)], srcs[b])
            pltpu.async_copy(sup_hbm.at[srcs[b]], rows[b], gsem[b])
            pltpu.async_copy(w_hbm.at[pl.ds(eb + b * C, C)],
                             wvs[b].at[pl.ds(0, C)], wsem[b])

        def chunk_step(kk, carry):
            for b in range(2):
                k = kk * 2 + b
                pltpu.make_async_copy(sup_hbm.at[srcs[b]], rows[b],
                                      gsem[b]).wait()

                @pl.when(kk < nkk - 1)
                def _():
                    pltpu.async_copy(src_hbm.at[pl.ds(eb + (k + 2) * C, C)],
                                     srcs[b], rsem[b])

                @pl.when(kk >= 1)
                def _():
                    pltpu.make_async_copy(
                        scaled[b], acc.at[dsts[b]], ssem[b]).wait()
                pltpu.async_copy(dst_hbm.at[pl.ds(eb + k * C, C)],
                                 dsts[b], dsem[b])

                pltpu.make_async_copy(w_hbm.at[pl.ds(eb + k * C, C)],
                                      wvs[b].at[pl.ds(0, C)], wsem[b]).wait()

                def edge(e, ecarry):
                    we = wvs[b][pl.ds(e, LANES)][0]
                    for fb in range(fb_n):
                        sl = pl.ds(fb * LANES, LANES)
                        scaled[b][e, sl] = rows[b][e, sl] * we
                    return ecarry
                lax.fori_loop(0, C, edge, 0)

                @pl.when(kk < nkk - 1)
                def _():
                    pltpu.make_async_copy(
                        src_hbm.at[pl.ds(eb + (k + 2) * C, C)], srcs[b],
                        rsem[b]).wait()
                    pltpu.async_copy(sup_hbm.at[srcs[b]], rows[b], gsem[b])
                    pltpu.async_copy(w_hbm.at[pl.ds(eb + (k + 2) * C, C)],
                                     wvs[b].at[pl.ds(0, C)], wsem[b])
                pltpu.make_async_copy(
                    dst_hbm.at[pl.ds(eb + k * C, C)], dsts[b],
                    dsem[b]).wait()
                pltpu.async_copy(scaled[b], acc.at[dsts[b]], ssem[b],
                                 add=True)
            return carry
        lax.fori_loop(0, nkk, chunk_step, 0)

        for b in range(2):
            pltpu.make_async_copy(scaled[b], acc.at[dsts[b]], ssem[b]).wait()
        plsc.subcore_barrier()
        r0 = sid * full

        @pl.when(sid < NS - 1)
        def _():
            pltpu.sync_copy(acc.at[pl.ds(r0, full)],
                            out_hbm.at[cid, pl.ds(r0, full)])

        @pl.when(sid == NS - 1)
        def _():
            pltpu.sync_copy(acc.at[pl.ds(r0, last)],
                            out_hbm.at[cid, pl.ds(r0, last)])

    return spmm(src, dst, ew, support)


def kernel(x, edge_index, edge_weight, W1, b1, W2, b2):
    n, _ = x.shape
    e = edge_weight.shape[0]

    quantum = NW * C * 2
    e_pad = ((e + quantum - 1) // quantum) * quantum
    pad = e_pad - e
    src = jnp.concatenate([edge_index[0], jnp.zeros((pad,), jnp.int32)])
    dst = jnp.concatenate([edge_index[1], jnp.zeros((pad,), jnp.int32)])
    ew = jnp.concatenate([edge_weight, jnp.zeros((pad,), jnp.float32)])

    support = _mm_tc(x, W1)
    p1 = _sc_spmm(src, dst, ew, support, n)
    s2 = _bias_relu_mm_tc(p1, b1, W2)
    p2 = _sc_spmm(src, dst, ew, s2, n)
    return _bias_relu_tc(p2, b2)


# speedup vs baseline: 5.6570x; 1.1195x over previous
import functools

import jax
import jax.numpy as jnp
from jax import lax
from jax.experimental import pallas as pl
from jax.experimental.pallas import tpu as pltpu
from jax.experimental.pallas import tpu_sc as plsc

NC = 2
NS = 16
NW = NC * NS
LANES = 16
C = 96
CORE1_SHARE = 27


def _mm_tc(x, w, block_rows=1000):
    n, kdim = x.shape
    m = w.shape[1]

    def body(x_ref, w_ref, o_ref):
        o_ref[...] = jnp.dot(x_ref[...], w_ref[...],
                             preferred_element_type=jnp.float32)

    return pl.pallas_call(
        body,
        grid=(n // block_rows,),
        in_specs=[pl.BlockSpec((block_rows, kdim), lambda i: (i, 0)),
                  pl.BlockSpec((kdim, m), lambda i: (0, 0))],
        out_specs=pl.BlockSpec((block_rows, m), lambda i: (i, 0)),
        out_shape=jax.ShapeDtypeStruct((n, m), jnp.float32),
    )(x, w)


def _bias_relu_mm_tc(parts, b, w, block_rows=1000):
    _, n, kdim = parts.shape
    m = w.shape[1]

    def body(p_ref, b_ref, w_ref, o_ref):
        h = jnp.maximum(p_ref[0] + p_ref[1] + b_ref[...], 0.0)
        o_ref[...] = jnp.dot(h, w_ref[...], preferred_element_type=jnp.float32)

    return pl.pallas_call(
        body,
        grid=(n // block_rows,),
        in_specs=[pl.BlockSpec((2, block_rows, kdim), lambda i: (0, i, 0)),
                  pl.BlockSpec((1, kdim), lambda i: (0, 0)),
                  pl.BlockSpec((kdim, m), lambda i: (0, 0))],
        out_specs=pl.BlockSpec((block_rows, m), lambda i: (i, 0)),
        out_shape=jax.ShapeDtypeStruct((n, m), jnp.float32),
    )(parts, b.reshape(1, kdim), w)


def _bias_relu_tc(parts, b, block_rows=1000):
    _, n, mp = parts.shape
    m = b.shape[0]

    def body(p_ref, b_ref, o_ref):
        o_ref[...] = jnp.maximum(
            p_ref[0, :, :m] + p_ref[1, :, :m] + b_ref[...], 0.0)

    return pl.pallas_call(
        body,
        grid=(n // block_rows,),
        in_specs=[pl.BlockSpec((2, block_rows, mp), lambda i: (0, i, 0)),
                  pl.BlockSpec((1, m), lambda i: (0, 0))],
        out_specs=pl.BlockSpec((block_rows, m), lambda i: (i, 0)),
        out_shape=jax.ShapeDtypeStruct((n, m), jnp.float32),
    )(parts, b.reshape(1, m))


def _sc_spmm(src, dst, ew, support, n_out):
    e_pad = src.shape[0]
    f = support.shape[1]
    nchunk_all = e_pad // (NS * C)
    nck = [nchunk_all - nchunk_all * CORE1_SHARE // 100,
           nchunk_all * CORE1_SHARE // 100]
    nck = [2 * (nc // 2) for nc in nck]
    nck[0] = nchunk_all - nck[1]
    assert nck[0] % 2 == 0 and nck[0] > 0 and nck[1] > 0
    fb_n = f // LANES
    zrows = 80
    full = ((n_out + NS - 1) // NS + zrows - 1) // zrows * zrows
    last = n_out - (NS - 1) * full
    assert 0 < last <= full and last % zrows == 0

    mesh = plsc.VectorSubcoreMesh(core_axis_name="c", subcore_axis_name="s",
                                  num_cores=NC, num_subcores=NS)
    params = (None if f % 128 == 0
              else pltpu.CompilerParams(use_tc_tiling_on_sc=False))

    @functools.partial(
        pl.kernel,
        out_type=jax.ShapeDtypeStruct((NC, n_out, f), jnp.float32),
        mesh=mesh,
        compiler_params=params,
        scratch_types=[
            pltpu.VMEM((C, f), jnp.float32),
            pltpu.VMEM((C, f), jnp.float32),
            pltpu.VMEM((C, f), jnp.float32),
            pltpu.VMEM((C, f), jnp.float32),
            pltpu.VMEM((C,), jnp.int32),
            pltpu.VMEM((C,), jnp.int32),
            pltpu.VMEM((C,), jnp.int32),
            pltpu.VMEM((C,), jnp.int32),
            pltpu.VMEM((C + LANES,), jnp.float32),
            pltpu.VMEM((C + LANES,), jnp.float32),
            pltpu.VMEM_SHARED((n_out, f), jnp.float32),
            pltpu.SemaphoreType.DMA,
            pltpu.SemaphoreType.DMA,
            pltpu.SemaphoreType.DMA,
            pltpu.SemaphoreType.DMA,
            pltpu.SemaphoreType.DMA,
            pltpu.SemaphoreType.DMA,
            pltpu.SemaphoreType.DMA,
            pltpu.SemaphoreType.DMA,
            pltpu.SemaphoreType.DMA,
            pltpu.SemaphoreType.DMA,
        ],
    )
    def spmm(src_hbm, dst_hbm, w_hbm, sup_hbm, out_hbm,
             rows0, rows1, sc0, sc1, si0, si1, di0, di1, wv0, wv1,
             acc, g0, g1, s0, s1, d0, d1, r0sem, r1sem, w0, w1):
        cid = lax.axis_index("c")
        sid = lax.axis_index("s")
        eb = jnp.where(cid == 0,
                       sid * (C * nck[0]),
                       NS * C * nck[0] + sid * (C * nck[1]))
        nkk = jnp.where(cid == 0, nck[0] // 2, nck[1] // 2)
        rows = (rows0, rows1)
        scaled = (sc0, sc1)
        srcs = (si0, si1)
        dsts = (di0, di1)
        wvs = (wv0, wv1)
        gsem = (g0, g1)
        ssem = (s0, s1)
        dsem = (d0, d1)
        rsem = (r0sem, r1sem)
        wsem = (w0, w1)

        def zrow(i, carry):
            for fb in range(fb_n):
                rows0[i, pl.ds(fb * LANES, LANES)] = jnp.zeros((LANES,),
                                                               jnp.float32)
            return carry
        lax.fori_loop(0, zrows, zrow, 0)
        for j in range(full // zrows):
            @pl.when(jnp.logical_or(sid < NS - 1, j < last // zrows))
            def _():
                pltpu.sync_copy(
                    rows0.at[pl.ds(0, zrows)],
                    acc.at[pl.ds(sid * full + j * zrows, zrows)])
        plsc.subcore_barrier()

        for b in range(2):
            pltpu.sync_copy(src_hbm.at[pl.ds(eb + b * C, C)], srcs[b])
            pltpu.async_copy(sup_hbm.at[srcs[b]], rows[b], gsem[b])
            pltpu.async_copy(w_hbm.at[pl.ds(eb + b * C, C)],
                             wvs[b].at[pl.ds(0, C)], wsem[b])

        def chunk_step(kk, carry):
            for b in range(2):
                k = kk * 2 + b
                pltpu.make_async_copy(sup_hbm.at[srcs[b]], rows[b],
                                      gsem[b]).wait()

                @pl.when(kk < nkk - 1)
                def _():
                    pltpu.async_copy(src_hbm.at[pl.ds(eb + (k + 2) * C, C)],
                                     srcs[b], rsem[b])

                @pl.when(kk >= 1)
                def _():
                    pltpu.make_async_copy(
                        scaled[b], acc.at[dsts[b]], ssem[b]).wait()
                pltpu.async_copy(dst_hbm.at[pl.ds(eb + k * C, C)],
                                 dsts[b], dsem[b])

                pltpu.make_async_copy(w_hbm.at[pl.ds(eb + k * C, C)],
                                      wvs[b].at[pl.ds(0, C)], wsem[b]).wait()

                def edge(e, ecarry):
                    we = wvs[b][pl.ds(e, LANES)][0]
                    for fb in range(fb_n):
                        sl = pl.ds(fb * LANES, LANES)
                        scaled[b][e, sl] = rows[b][e, sl] * we
                    return ecarry
                lax.fori_loop(0, C, edge, 0)

                @pl.when(kk < nkk - 1)
                def _():
                    pltpu.make_async_copy(
                        src_hbm.at[pl.ds(eb + (k + 2) * C, C)], srcs[b],
                        rsem[b]).wait()
                    pltpu.async_copy(sup_hbm.at[srcs[b]], rows[b], gsem[b])
                    pltpu.async_copy(w_hbm.at[pl.ds(eb + (k + 2) * C, C)],
                                     wvs[b].at[pl.ds(0, C)], wsem[b])
                pltpu.make_async_copy(
                    dst_hbm.at[pl.ds(eb + k * C, C)], dsts[b],
                    dsem[b]).wait()
                pltpu.async_copy(scaled[b], acc.at[dsts[b]], ssem[b],
                                 add=True)
            return carry
        lax.fori_loop(0, nkk, chunk_step, 0)

        for b in range(2):
            pltpu.make_async_copy(scaled[b], acc.at[dsts[b]], ssem[b]).wait()
        plsc.subcore_barrier()
        r0 = sid * full

        @pl.when(sid < NS - 1)
        def _():
            pltpu.sync_copy(acc.at[pl.ds(r0, full)],
                            out_hbm.at[cid, pl.ds(r0, full)])

        @pl.when(sid == NS - 1)
        def _():
            pltpu.sync_copy(acc.at[pl.ds(r0, last)],
                            out_hbm.at[cid, pl.ds(r0, last)])

    return spmm(src, dst, ew, support)


def kernel(x, edge_index, edge_weight, W1, b1, W2, b2):
    n, _ = x.shape
    e = edge_weight.shape[0]

    quantum = NW * C * 2
    e_pad = ((e + quantum - 1) // quantum) * quantum
    pad = e_pad - e
    src = jnp.concatenate([edge_index[0], jnp.zeros((pad,), jnp.int32)])
    dst = jnp.concatenate([edge_index[1], jnp.zeros((pad,), jnp.int32)])
    ew = jnp.concatenate([edge_weight, jnp.zeros((pad,), jnp.float32)])

    support = _mm_tc(x, W1)
    p1 = _sc_spmm(src, dst, ew, support, n)
    s2 = _bias_relu_mm_tc(p1, b1, W2)
    p2 = _sc_spmm(src, dst, ew, s2, n)
    return _bias_relu_tc(p2, b2)


# speedup vs baseline: 10.6509x; 1.8828x over previous
import functools

import jax
import jax.numpy as jnp
from jax import lax
from jax.experimental import pallas as pl
from jax.experimental.pallas import tpu as pltpu
from jax.experimental.pallas import tpu_sc as plsc

NC = 2
NS = 16
NW = NC * NS
LANES = 16
C = 96
CORE1_SHARE = 50


def _mm_tc(x, w, block_rows=1000):
    n, kdim = x.shape
    m = w.shape[1]

    def body(x_ref, w_ref, o_ref):
        o_ref[...] = jnp.dot(x_ref[...], w_ref[...],
                             preferred_element_type=jnp.float32)

    return pl.pallas_call(
        body,
        grid=(n // block_rows,),
        in_specs=[pl.BlockSpec((block_rows, kdim), lambda i: (i, 0)),
                  pl.BlockSpec((kdim, m), lambda i: (0, 0))],
        out_specs=pl.BlockSpec((block_rows, m), lambda i: (i, 0)),
        out_shape=jax.ShapeDtypeStruct((n, m), jnp.float32),
    )(x, w)


def _bias_relu_mm_tc(parts, b, w, block_rows=1000):
    _, n, kdim = parts.shape
    m = w.shape[1]

    def body(p_ref, b_ref, w_ref, o_ref):
        h = jnp.maximum(p_ref[0] + p_ref[1] + b_ref[...], 0.0)
        o_ref[...] = jnp.dot(h, w_ref[...], preferred_element_type=jnp.float32)

    return pl.pallas_call(
        body,
        grid=(n // block_rows,),
        in_specs=[pl.BlockSpec((2, block_rows, kdim), lambda i: (0, i, 0)),
                  pl.BlockSpec((1, kdim), lambda i: (0, 0)),
                  pl.BlockSpec((kdim, m), lambda i: (0, 0))],
        out_specs=pl.BlockSpec((block_rows, m), lambda i: (i, 0)),
        out_shape=jax.ShapeDtypeStruct((n, m), jnp.float32),
    )(parts, b.reshape(1, kdim), w)


def _bias_relu_tc(parts, b, block_rows=1000):
    _, n, mp = parts.shape
    m = b.shape[0]

    def body(p_ref, b_ref, o_ref):
        o_ref[...] = jnp.maximum(
            p_ref[0, :, :m] + p_ref[1, :, :m] + b_ref[...], 0.0)

    return pl.pallas_call(
        body,
        grid=(n // block_rows,),
        in_specs=[pl.BlockSpec((2, block_rows, mp), lambda i: (0, i, 0)),
                  pl.BlockSpec((1, m), lambda i: (0, 0))],
        out_specs=pl.BlockSpec((block_rows, m), lambda i: (i, 0)),
        out_shape=jax.ShapeDtypeStruct((n, m), jnp.float32),
    )(parts, b.reshape(1, m))


def _sc_spmm(src, dst, ew, support, n_out):
    e_pad = src.shape[0]
    f = support.shape[1]
    nchunk_all = e_pad // (NS * C)
    nck = [nchunk_all - nchunk_all * CORE1_SHARE // 100,
           nchunk_all * CORE1_SHARE // 100]
    nck = [2 * (nc // 2) for nc in nck]
    nck[0] = nchunk_all - nck[1]
    assert nck[0] % 2 == 0 and nck[0] > 0 and nck[1] > 0
    fb_n = f // LANES
    zrows = 80
    full = ((n_out + NS - 1) // NS + zrows - 1) // zrows * zrows
    last = n_out - (NS - 1) * full
    assert 0 < last <= full and last % zrows == 0

    mesh = plsc.VectorSubcoreMesh(core_axis_name="c", subcore_axis_name="s",
                                  num_cores=NC, num_subcores=NS)
    params = (None if f % 128 == 0
              else pltpu.CompilerParams(use_tc_tiling_on_sc=False))

    @functools.partial(
        pl.kernel,
        out_type=jax.ShapeDtypeStruct((NC, n_out, f), jnp.float32),
        mesh=mesh,
        compiler_params=params,
        scratch_types=[
            pltpu.VMEM((C, f), jnp.float32),
            pltpu.VMEM((C, f), jnp.float32),
            pltpu.VMEM((C, f), jnp.float32),
            pltpu.VMEM((C, f), jnp.float32),
            pltpu.VMEM((C,), jnp.int32),
            pltpu.VMEM((C,), jnp.int32),
            pltpu.VMEM((C,), jnp.int32),
            pltpu.VMEM((C,), jnp.int32),
            pltpu.VMEM((C + LANES,), jnp.float32),
            pltpu.VMEM((C + LANES,), jnp.float32),
            pltpu.VMEM_SHARED((n_out, f), jnp.float32),
            pltpu.SemaphoreType.DMA,
            pltpu.SemaphoreType.DMA,
            pltpu.SemaphoreType.DMA,
            pltpu.SemaphoreType.DMA,
            pltpu.SemaphoreType.DMA,
            pltpu.SemaphoreType.DMA,
            pltpu.SemaphoreType.DMA,
            pltpu.SemaphoreType.DMA,
            pltpu.SemaphoreType.DMA,
            pltpu.SemaphoreType.DMA,
        ],
    )
    def spmm(src_hbm, dst_hbm, w_hbm, sup_hbm, out_hbm,
             rows0, rows1, sc0, sc1, si0, si1, di0, di1, wv0, wv1,
             acc, g0, g1, s0, s1, d0, d1, r0sem, r1sem, w0, w1):
        cid = lax.axis_index("c")
        sid = lax.axis_index("s")
        eb = jnp.where(cid == 0,
                       sid * (C * nck[0]),
                       NS * C * nck[0] + sid * (C * nck[1]))
        nkk = jnp.where(cid == 0, nck[0] // 2, nck[1] // 2)
        rows = (rows0, rows1)
        scaled = (sc0, sc1)
        srcs = (si0, si1)
        dsts = (di0, di1)
        wvs = (wv0, wv1)
        gsem = (g0, g1)
        ssem = (s0, s1)
        dsem = (d0, d1)
        rsem = (r0sem, r1sem)
        wsem = (w0, w1)

        def zrow(i, carry):
            for fb in range(fb_n):
                rows0[i, pl.ds(fb * LANES, LANES)] = jnp.zeros((LANES,),
                                                               jnp.float32)
            return carry
        lax.fori_loop(0, zrows, zrow, 0)
        for j in range(full // zrows):
            @pl.when(jnp.logical_or(sid < NS - 1, j < last // zrows))
            def _():
                pltpu.sync_copy(
                    rows0.at[pl.ds(0, zrows)],
                    acc.at[pl.ds(sid * full + j * zrows, zrows)])
        plsc.subcore_barrier()

        for b in range(2):
            pltpu.sync_copy(src_hbm.at[pl.ds(eb + b * C, C)], srcs[b])
            pltpu.async_copy(sup_hbm.at[srcs[b]], rows[b], gsem[b])
            pltpu.async_copy(w_hbm.at[pl.ds(eb + b * C, C)],
                             wvs[b].at[pl.ds(0, C)], wsem[b])

        def chunk_step(kk, carry):
            for b in range(2):
                k = kk * 2 + b
                pltpu.make_async_copy(sup_hbm.at[srcs[b]], rows[b],
                                      gsem[b]).wait()

                @pl.when(kk < nkk - 1)
                def _():
                    pltpu.async_copy(src_hbm.at[pl.ds(eb + (k + 2) * C, C)],
                                     srcs[b], rsem[b])

                @pl.when(kk >= 1)
                def _():
                    pltpu.make_async_copy(
                        scaled[b], acc.at[dsts[b]], ssem[b]).wait()
                pltpu.async_copy(dst_hbm.at[pl.ds(eb + k * C, C)],
                                 dsts[b], dsem[b])

                pltpu.make_async_copy(w_hbm.at[pl.ds(eb + k * C, C)],
                                      wvs[b].at[pl.ds(0, C)], wsem[b]).wait()

                def edge(e, ecarry):
                    we = wvs[b][pl.ds(e, LANES)][0]
                    for fb in range(fb_n):
                        sl = pl.ds(fb * LANES, LANES)
                        scaled[b][e, sl] = rows[b][e, sl] * we
                    return ecarry
                lax.fori_loop(0, C, edge, 0)

                @pl.when(kk < nkk - 1)
                def _():
                    pltpu.make_async_copy(
                        src_hbm.at[pl.ds(eb + (k + 2) * C, C)], srcs[b],
                        rsem[b]).wait()
                    pltpu.async_copy(sup_hbm.at[srcs[b]], rows[b], gsem[b])
                    pltpu.async_copy(w_hbm.at[pl.ds(eb + (k + 2) * C, C)],
                                     wvs[b].at[pl.ds(0, C)], wsem[b])
                pltpu.make_async_copy(
                    dst_hbm.at[pl.ds(eb + k * C, C)], dsts[b],
                    dsem[b]).wait()
                pltpu.async_copy(scaled[b], acc.at[dsts[b]], ssem[b],
                                 add=True)
            return carry
        lax.fori_loop(0, nkk, chunk_step, 0)

        for b in range(2):
            pltpu.make_async_copy(scaled[b], acc.at[dsts[b]], ssem[b]).wait()
        plsc.subcore_barrier()
        r0 = sid * full

        @pl.when(sid < NS - 1)
        def _():
            pltpu.sync_copy(acc.at[pl.ds(r0, full)],
                            out_hbm.at[cid, pl.ds(r0, full)])

        @pl.when(sid == NS - 1)
        def _():
            pltpu.sync_copy(acc.at[pl.ds(r0, last)],
                            out_hbm.at[cid, pl.ds(r0, last)])

    return spmm(src, dst, ew, support)


def kernel(x, edge_index, edge_weight, W1, b1, W2, b2):
    n, _ = x.shape
    e = edge_weight.shape[0]

    quantum = NW * C * 2
    e_pad = ((e + quantum - 1) // quantum) * quantum
    pad = e_pad - e
    spread = jnp.arange(pad, dtype=jnp.int32) % jnp.int32(n)
    src = jnp.concatenate([edge_index[0], spread])
    dst = jnp.concatenate([edge_index[1], spread])
    ew = jnp.concatenate([edge_weight, jnp.zeros((pad,), jnp.float32)])

    support = _mm_tc(x, W1)
    p1 = _sc_spmm(src, dst, ew, support, n)
    s2 = _bias_relu_mm_tc(p1, b1, W2)
    p2 = _sc_spmm(src, dst, ew, s2, n)
    return _bias_relu_tc(p2, b2)


# speedup vs baseline: 11.1727x; 1.0490x over previous
import functools

import jax
import jax.numpy as jnp
from jax import lax
from jax.experimental import pallas as pl
from jax.experimental.pallas import tpu as pltpu
from jax.experimental.pallas import tpu_sc as plsc

NC = 2
NS = 16
NW = NC * NS
LANES = 16
C = 96
CORE1_SHARE = 50


def _mm_tc(x, w, block_rows=1000):
    n, kdim = x.shape
    m = w.shape[1]

    def body(x_ref, w_ref, o_ref):
        o_ref[...] = jnp.dot(x_ref[...], w_ref[...],
                             preferred_element_type=jnp.float32)

    return pl.pallas_call(
        body,
        grid=(n // block_rows,),
        in_specs=[pl.BlockSpec((block_rows, kdim), lambda i: (i, 0)),
                  pl.BlockSpec((kdim, m), lambda i: (0, 0))],
        out_specs=pl.BlockSpec((block_rows, m), lambda i: (i, 0)),
        out_shape=jax.ShapeDtypeStruct((n, m), jnp.float32),
    )(x, w)


def _bias_relu_mm_tc(parts, b, w, block_rows=1000):
    _, n, kdim = parts.shape
    m = w.shape[1]

    def body(p_ref, b_ref, w_ref, o_ref):
        h = jnp.maximum(p_ref[0] + p_ref[1] + b_ref[...], 0.0)
        o_ref[...] = jnp.dot(h, w_ref[...], preferred_element_type=jnp.float32)

    return pl.pallas_call(
        body,
        grid=(n // block_rows,),
        in_specs=[pl.BlockSpec((2, block_rows, kdim), lambda i: (0, i, 0)),
                  pl.BlockSpec((1, kdim), lambda i: (0, 0)),
                  pl.BlockSpec((kdim, m), lambda i: (0, 0))],
        out_specs=pl.BlockSpec((block_rows, m), lambda i: (i, 0)),
        out_shape=jax.ShapeDtypeStruct((n, m), jnp.float32),
    )(parts, b.reshape(1, kdim), w)


def _bias_relu_tc(parts, b, block_rows=1000):
    _, n, mp = parts.shape
    m = b.shape[0]

    def body(p_ref, b_ref, o_ref):
        o_ref[...] = jnp.maximum(
            p_ref[0, :, :m] + p_ref[1, :, :m] + b_ref[...], 0.0)

    return pl.pallas_call(
        body,
        grid=(n // block_rows,),
        in_specs=[pl.BlockSpec((2, block_rows, mp), lambda i: (0, i, 0)),
                  pl.BlockSpec((1, m), lambda i: (0, 0))],
        out_specs=pl.BlockSpec((block_rows, m), lambda i: (i, 0)),
        out_shape=jax.ShapeDtypeStruct((n, m), jnp.float32),
    )(parts, b.reshape(1, m))


def _sc_spmm(src, dst, ew, support, n_out):
    e_pad = src.shape[0]
    f = support.shape[1]
    nchunk_all = e_pad // (NS * C)
    nck = [nchunk_all - nchunk_all * CORE1_SHARE // 100,
           nchunk_all * CORE1_SHARE // 100]
    nck = [2 * (nc // 2) for nc in nck]
    nck[0] = nchunk_all - nck[1]
    assert nck[0] % 2 == 0 and nck[0] > 0 and nck[1] > 0
    fb_n = f // LANES
    zrows = 80
    full = ((n_out + NS - 1) // NS + zrows - 1) // zrows * zrows
    last = n_out - (NS - 1) * full
    assert 0 < last <= full and last % zrows == 0

    mesh = plsc.VectorSubcoreMesh(core_axis_name="c", subcore_axis_name="s",
                                  num_cores=NC, num_subcores=NS)
    params = (None if f % 128 == 0
              else pltpu.CompilerParams(use_tc_tiling_on_sc=False))

    @functools.partial(
        pl.kernel,
        out_type=jax.ShapeDtypeStruct((NC, n_out, f), jnp.float32),
        mesh=mesh,
        compiler_params=params,
        scratch_types=[
            pltpu.VMEM((C, f), jnp.float32),
            pltpu.VMEM((C, f), jnp.float32),
            pltpu.VMEM((C, f), jnp.float32),
            pltpu.VMEM((C, f), jnp.float32),
            pltpu.VMEM((C,), jnp.int32),
            pltpu.VMEM((C,), jnp.int32),
            pltpu.VMEM((C,), jnp.int32),
            pltpu.VMEM((C,), jnp.int32),
            pltpu.VMEM((C + LANES,), jnp.float32),
            pltpu.VMEM((C + LANES,), jnp.float32),
            pltpu.VMEM_SHARED((n_out, f), jnp.float32),
            pltpu.SemaphoreType.DMA,
            pltpu.SemaphoreType.DMA,
            pltpu.SemaphoreType.DMA,
            pltpu.SemaphoreType.DMA,
            pltpu.SemaphoreType.DMA,
            pltpu.SemaphoreType.DMA,
            pltpu.SemaphoreType.DMA,
            pltpu.SemaphoreType.DMA,
            pltpu.SemaphoreType.DMA,
            pltpu.SemaphoreType.DMA,
        ],
    )
    def spmm(src_hbm, dst_hbm, w_hbm, sup_hbm, out_hbm,
             rows0, rows1, sc0, sc1, si0, si1, di0, di1, wv0, wv1,
             acc, g0, g1, s0, s1, d0, d1, r0sem, r1sem, w0, w1):
        cid = lax.axis_index("c")
        sid = lax.axis_index("s")
        eb = jnp.where(cid == 0,
                       sid * (C * nck[0]),
                       NS * C * nck[0] + sid * (C * nck[1]))
        nkk = jnp.where(cid == 0, nck[0] // 2, nck[1] // 2)
        rows = (rows0, rows1)
        scaled = (sc0, sc1)
        srcs = (si0, si1)
        dsts = (di0, di1)
        wvs = (wv0, wv1)
        gsem = (g0, g1)
        ssem = (s0, s1)
        dsem = (d0, d1)
        rsem = (r0sem, r1sem)
        wsem = (w0, w1)

        def zrow(i, carry):
            for fb in range(fb_n):
                rows0[i, pl.ds(fb * LANES, LANES)] = jnp.zeros((LANES,),
                                                               jnp.float32)
            return carry
        lax.fori_loop(0, zrows, zrow, 0)
        for j in range(full // zrows):
            @pl.when(jnp.logical_or(sid < NS - 1, j < last // zrows))
            def _():
                pltpu.sync_copy(
                    rows0.at[pl.ds(0, zrows)],
                    acc.at[pl.ds(sid * full + j * zrows, zrows)])
        plsc.subcore_barrier()

        for b in range(2):
            pltpu.sync_copy(src_hbm.at[pl.ds(eb + b * C, C)], srcs[b])
            pltpu.async_copy(sup_hbm.at[srcs[b]], rows[b], gsem[b])
            pltpu.async_copy(w_hbm.at[pl.ds(eb + b * C, C)],
                             wvs[b].at[pl.ds(0, C)], wsem[b])

        def chunk_step(kk, carry):
            for b in range(2):
                k = kk * 2 + b
                pltpu.make_async_copy(sup_hbm.at[srcs[b]], rows[b],
                                      gsem[b]).wait()

                @pl.when(kk < nkk - 1)
                def _():
                    pltpu.async_copy(src_hbm.at[pl.ds(eb + (k + 2) * C, C)],
                                     srcs[b], rsem[b])

                @pl.when(kk >= 1)
                def _():
                    pltpu.make_async_copy(
                        scaled[b], acc.at[dsts[b]], ssem[b]).wait()
                pltpu.async_copy(dst_hbm.at[pl.ds(eb + k * C, C)],
                                 dsts[b], dsem[b])

                pltpu.make_async_copy(w_hbm.at[pl.ds(eb + k * C, C)],
                                      wvs[b].at[pl.ds(0, C)], wsem[b]).wait()

                @plsc.parallel_loop(0, C, step=1, unroll=4)
                def edge(e):
                    we = wvs[b][pl.ds(e, LANES)][0]
                    for fb in range(fb_n):
                        sl = pl.ds(fb * LANES, LANES)
                        scaled[b][e, sl] = rows[b][e, sl] * we

                @pl.when(kk < nkk - 1)
                def _():
                    pltpu.make_async_copy(
                        src_hbm.at[pl.ds(eb + (k + 2) * C, C)], srcs[b],
                        rsem[b]).wait()
                    pltpu.async_copy(sup_hbm.at[srcs[b]], rows[b], gsem[b])
                    pltpu.async_copy(w_hbm.at[pl.ds(eb + (k + 2) * C, C)],
                                     wvs[b].at[pl.ds(0, C)], wsem[b])
                pltpu.make_async_copy(
                    dst_hbm.at[pl.ds(eb + k * C, C)], dsts[b],
                    dsem[b]).wait()
                pltpu.async_copy(scaled[b], acc.at[dsts[b]], ssem[b],
                                 add=True)
            return carry
        lax.fori_loop(0, nkk, chunk_step, 0)

        for b in range(2):
            pltpu.make_async_copy(scaled[b], acc.at[dsts[b]], ssem[b]).wait()
        plsc.subcore_barrier()
        r0 = sid * full

        @pl.when(sid < NS - 1)
        def _():
            pltpu.sync_copy(acc.at[pl.ds(r0, full)],
                            out_hbm.at[cid, pl.ds(r0, full)])

        @pl.when(sid == NS - 1)
        def _():
            pltpu.sync_copy(acc.at[pl.ds(r0, last)],
                            out_hbm.at[cid, pl.ds(r0, last)])

    return spmm(src, dst, ew, support)


def kernel(x, edge_index, edge_weight, W1, b1, W2, b2):
    n, _ = x.shape
    e = edge_weight.shape[0]

    quantum = NW * C * 2
    e_pad = ((e + quantum - 1) // quantum) * quantum
    pad = e_pad - e
    spread = jnp.arange(pad, dtype=jnp.int32) % jnp.int32(n)
    src = jnp.concatenate([edge_index[0], spread])
    dst = jnp.concatenate([edge_index[1], spread])
    ew = jnp.concatenate([edge_weight, jnp.zeros((pad,), jnp.float32)])

    support = _mm_tc(x, W1)
    p1 = _sc_spmm(src, dst, ew, support, n)
    s2 = _bias_relu_mm_tc(p1, b1, W2)
    p2 = _sc_spmm(src, dst, ew, s2, n)
    return _bias_relu_tc(p2, b2)


# speedup vs baseline: 11.5846x; 1.0369x over previous
import functools

import jax
import jax.numpy as jnp
from jax import lax
from jax.experimental import pallas as pl
from jax.experimental.pallas import tpu as pltpu
from jax.experimental.pallas import tpu_sc as plsc

NC = 2
NS = 16
NW = NC * NS
LANES = 16
C = 128
RING = 3


def _mm_tc(x, w, block_rows=1000):
    n, kdim = x.shape
    m = w.shape[1]

    def body(x_ref, w_ref, o_ref):
        o_ref[...] = jnp.dot(x_ref[...], w_ref[...],
                             preferred_element_type=jnp.float32)

    return pl.pallas_call(
        body,
        grid=(n // block_rows,),
        in_specs=[pl.BlockSpec((block_rows, kdim), lambda i: (i, 0)),
                  pl.BlockSpec((kdim, m), lambda i: (0, 0))],
        out_specs=pl.BlockSpec((block_rows, m), lambda i: (i, 0)),
        out_shape=jax.ShapeDtypeStruct((n, m), jnp.float32),
    )(x, w)


def _bias_relu_mm_tc(parts, b, w, block_rows=1000):
    _, n, kdim = parts.shape
    m = w.shape[1]

    def body(p_ref, b_ref, w_ref, o_ref):
        h = jnp.maximum(p_ref[0] + p_ref[1] + b_ref[...], 0.0)
        o_ref[...] = jnp.dot(h, w_ref[...], preferred_element_type=jnp.float32)

    return pl.pallas_call(
        body,
        grid=(n // block_rows,),
        in_specs=[pl.BlockSpec((2, block_rows, kdim), lambda i: (0, i, 0)),
                  pl.BlockSpec((1, kdim), lambda i: (0, 0)),
                  pl.BlockSpec((kdim, m), lambda i: (0, 0))],
        out_specs=pl.BlockSpec((block_rows, m), lambda i: (i, 0)),
        out_shape=jax.ShapeDtypeStruct((n, m), jnp.float32),
    )(parts, b.reshape(1, kdim), w)


def _bias_relu_tc(parts, b, block_rows=1000):
    _, n, m = parts.shape

    def body(p_ref, b_ref, o_ref):
        o_ref[...] = jnp.maximum(p_ref[0] + p_ref[1] + b_ref[...], 0.0)

    return pl.pallas_call(
        body,
        grid=(n // block_rows,),
        in_specs=[pl.BlockSpec((2, block_rows, m), lambda i: (0, i, 0)),
                  pl.BlockSpec((1, m), lambda i: (0, 0))],
        out_specs=pl.BlockSpec((block_rows, m), lambda i: (i, 0)),
        out_shape=jax.ShapeDtypeStruct((n, m), jnp.float32),
    )(parts, b.reshape(1, m))


def _sc_spmm(pk, dst, support, n_out):
    nchunks, _, c = pk.shape
    assert c == C
    f = support.shape[1]
    ncw = nchunks // NW
    nkk = ncw // RING
    fb_n = f // LANES
    zrows = 80
    full = ((n_out + NS - 1) // NS + zrows - 1) // zrows * zrows
    last = n_out - (NS - 1) * full
    assert 0 < last <= full and last % zrows == 0

    mesh = plsc.VectorSubcoreMesh(core_axis_name="c", subcore_axis_name="s",
                                  num_cores=NC, num_subcores=NS)
    params = (None if f % 128 == 0
              else pltpu.CompilerParams(use_tc_tiling_on_sc=False))

    @functools.partial(
        pl.kernel,
        out_type=jax.ShapeDtypeStruct((NC, n_out, f), jnp.float32),
        mesh=mesh,
        compiler_params=params,
        scratch_types=(
            [pltpu.VMEM((C, f), jnp.float32) for _ in range(RING)]
            + [pltpu.VMEM((2, C), jnp.int32) for _ in range(RING)]
            + [pltpu.VMEM((C,), jnp.int32) for _ in range(RING)]
            + [pltpu.VMEM_SHARED((n_out, f), jnp.float32)]
            + [pltpu.SemaphoreType.DMA] * (4 * RING)
        ),
    )
    def spmm(pk_hbm, dst_hbm, sup_hbm, out_hbm, *refs):
        rows = refs[0:RING]
        pkb = refs[RING:2 * RING]
        dsts = refs[2 * RING:3 * RING]
        acc = refs[3 * RING]
        gsem = refs[3 * RING + 1:3 * RING + 1 + RING]
        ssem = refs[3 * RING + 1 + RING:3 * RING + 1 + 2 * RING]
        dsem = refs[3 * RING + 1 + 2 * RING:3 * RING + 1 + 3 * RING]
        psem = refs[3 * RING + 1 + 3 * RING:3 * RING + 1 + 4 * RING]

        cid = lax.axis_index("c")
        sid = lax.axis_index("s")
        wid = cid * NS + sid
        cb = wid * ncw
        eb = cb * C

        def zrow(i, carry):
            for fb in range(fb_n):
                rows[0][i, pl.ds(fb * LANES, LANES)] = jnp.zeros(
                    (LANES,), jnp.float32)
            return carry
        lax.fori_loop(0, zrows, zrow, 0)
        for j in range(full // zrows):
            @pl.when(jnp.logical_or(sid < NS - 1, j < last // zrows))
            def _():
                pltpu.sync_copy(
                    rows[0].at[pl.ds(0, zrows)],
                    acc.at[pl.ds(sid * full + j * zrows, zrows)])
        plsc.subcore_barrier()

        for b in range(2):
            pltpu.sync_copy(pk_hbm.at[cb + b], pkb[b])
            pltpu.async_copy(sup_hbm.at[pkb[b].at[1]], rows[b], gsem[b])

        def chunk_step(kk, carry):
            for b in range(RING):
                k = kk * RING + b
                t2 = (b + 2) % RING
                more = k + 2 < ncw

                pltpu.make_async_copy(sup_hbm.at[pkb[b].at[1]], rows[b],
                                      gsem[b]).wait()

                @pl.when(more)
                def _():
                    pltpu.async_copy(pk_hbm.at[cb + k + 2], pkb[t2],
                                     psem[t2])
                pltpu.async_copy(dst_hbm.at[pl.ds(eb + k * C, C)],
                                 dsts[b], dsem[b])

                @plsc.parallel_loop(0, C, step=1, unroll=4)
                def edge(e):
                    wv = pkb[b][0, pl.ds(e, LANES)]
                    we = lax.bitcast_convert_type(wv[0], jnp.float32)
                    for fb in range(fb_n):
                        sl = pl.ds(fb * LANES, LANES)
                        rows[b][e, sl] = rows[b][e, sl] * we

                @pl.when(jnp.logical_and(k >= 1, more))
                def _():
                    pltpu.make_async_copy(rows[t2], acc.at[dsts[t2]],
                                          ssem[t2]).wait()

                @pl.when(more)
                def _():
                    pltpu.make_async_copy(pk_hbm.at[cb + k + 2], pkb[t2],
                                          psem[t2]).wait()
                    pltpu.async_copy(sup_hbm.at[pkb[t2].at[1]], rows[t2],
                                     gsem[t2])

                pltpu.make_async_copy(dst_hbm.at[pl.ds(eb + k * C, C)],
                                      dsts[b], dsem[b]).wait()
                pltpu.async_copy(rows[b], acc.at[dsts[b]], ssem[b],
                                 add=True)
            return carry
        lax.fori_loop(0, nkk, chunk_step, 0)

        for s in range(RING):
            pltpu.make_async_copy(rows[s], acc.at[dsts[s]], ssem[s]).wait()
        plsc.subcore_barrier()
        r0 = sid * full

        @pl.when(sid < NS - 1)
        def _():
            pltpu.sync_copy(acc.at[pl.ds(r0, full)],
                            out_hbm.at[cid, pl.ds(r0, full)])

        @pl.when(sid == NS - 1)
        def _():
            pltpu.sync_copy(acc.at[pl.ds(r0, last)],
                            out_hbm.at[cid, pl.ds(r0, last)])

    return spmm(pk, dst, support)


def kernel(x, edge_index, edge_weight, W1, b1, W2, b2):
    n, _ = x.shape
    e = edge_weight.shape[0]

    ncw = -(-e // (NW * C))
    ncw = ((ncw + RING - 1) // RING) * RING
    e_pad = NW * C * ncw
    pad = e_pad - e
    spread = jnp.arange(pad, dtype=jnp.int32) % jnp.int32(n)
    src = jnp.concatenate([edge_index[0], spread])
    dst = jnp.concatenate([edge_index[1], spread])
    ew = jnp.concatenate([edge_weight, jnp.zeros((pad,), jnp.float32)])
    wbits = lax.bitcast_convert_type(ew, jnp.int32)
    pk = jnp.stack([wbits.reshape(-1, C), src.reshape(-1, C)], axis=1)

    support = _mm_tc(x, W1)
    p1 = _sc_spmm(pk, dst, support, n)
    s2 = _bias_relu_mm_tc(p1, b1, W2)
    p2 = _sc_spmm(pk, dst, s2, n)
    return _bias_relu_tc(p2, b2)
